# baseline (speedup 1.0000x reference)
import functools

import jax
import jax.numpy as jnp
from jax import lax
from jax.experimental import pallas as pl
from jax.experimental.pallas import tpu as pltpu
from jax.experimental.pallas import tpu_sc as plsc

ROWS = 4096
COLS = 200
EMBED_DIM = 32
B = ROWS * COLS

_NUM_CORES = 2
_NUM_SUBCORES = 16
NW = _NUM_CORES * _NUM_SUBCORES

UNIT = 512
IHR = UNIT // 128
NDH = EMBED_DIM // 8
UNITS_PER_COL = ROWS // UNIT
NUNITS = COLS * UNITS_PER_COL
UNITS_PER_W = NUNITS // NW


def _make_gather():
    mesh = plsc.VectorSubcoreMesh(core_axis_name="c", subcore_axis_name="s")

    @functools.partial(
        pl.kernel,
        mesh=mesh,
        out_type=jax.ShapeDtypeStruct((COLS, NDH, ROWS // 128, 8, 128),
                                      jnp.float32),
        compiler_params=pltpu.CompilerParams(
            use_tc_tiling_on_sc=False, needs_layout_passes=False),
        scratch_types=[
            pltpu.VMEM((UNIT,), jnp.int32),
            pltpu.VMEM((UNIT,), jnp.int32),
            pltpu.VMEM((UNIT, EMBED_DIM), jnp.float32),
            pltpu.VMEM((UNIT, EMBED_DIM), jnp.float32),
            pltpu.VMEM((NDH, IHR, 8, 128), jnp.float32),
            pltpu.VMEM((NDH, IHR, 8, 128), jnp.float32),
            pltpu.SemaphoreType.DMA,
            pltpu.SemaphoreType.DMA,
            pltpu.SemaphoreType.DMA,
            pltpu.SemaphoreType.DMA,
        ],
    )
    def gather_kernel(idx_hbm, table_hbm, out5_hbm, idx0, idx1, g0, g1,
                      t0, t1, sg0, sg1, sw0, sw1):
        wid = lax.axis_index("s") * _NUM_CORES + lax.axis_index("c")
        u0 = wid * UNITS_PER_W
        idx_b = (idx0, idx1)
        g_b = (g0, g1)
        t_b = (t0, t1)
        sg = (sg0, sg1)
        sw = (sw0, sw1)
        iota = lax.iota(jnp.int32, 16)

        def idx_off(u):
            gu = u0 + u
            return (gu // UNITS_PER_COL) * ROWS + (gu % UNITS_PER_COL) * UNIT

        pltpu.sync_copy(idx_hbm.at[pl.ds(idx_off(0), UNIT)], idx0)
        pltpu.async_copy(table_hbm.at[idx0], g0, sg0)

        def unit_step(u, b):
            gu = u0 + u
            j = gu // UNITS_PER_COL
            ir = gu % UNITS_PER_COL
            gbuf = g_b[b]
            tbuf = t_b[b]

            pltpu.make_async_copy(table_hbm.at[idx_b[b]], gbuf, sg[b]).wait()

            @pl.when(u + 1 < UNITS_PER_W)
            def _prefetch():
                pltpu.sync_copy(
                    idx_hbm.at[pl.ds(idx_off(u + 1), UNIT)], idx_b[1 - b])
                pltpu.async_copy(
                    table_hbm.at[idx_b[1 - b]], g_b[1 - b], sg[1 - b])

            @pl.when(u >= 2)
            def _drain():
                for _ in range(NDH):
                    pltpu.make_async_copy(
                        tbuf.at[0], out5_hbm.at[0, 0, pl.ds(0, IHR)], sw[b]
                    ).wait()

            def transpose_step(k, _):
                dh = k // 32
                rem = k % 32
                ihr = rem // 8
                dl = rem % 8
                d = dh * 8 + dl
                dvec = jnp.full((16,), 0, jnp.int32) + d
                mbase = ihr * 128
                for il8 in range(8):
                    mvec = mbase + il8 * 16 + iota
                    vals = plsc.load_gather(gbuf, [mvec, dvec])
                    tbuf[dh, ihr, dl, pl.ds(il8 * 16, 16)] = vals
                return 0

            lax.fori_loop(0, NDH * 32, transpose_step, 0)

            for dh in range(NDH):
                pltpu.async_copy(
                    tbuf.at[dh],
                    out5_hbm.at[j, dh, pl.ds(ir * IHR, IHR)],
                    sw[b],
                )

        def outer(g2, _):
            for b in range(2):
                unit_step(g2 * 2 + b, b)
            return 0

        lax.fori_loop(0, UNITS_PER_W // 2, outer, 0)

        for b in range(2):
            for _ in range(NDH):
                pltpu.make_async_copy(
                    t_b[b].at[0], out5_hbm.at[0, 0, pl.ds(0, IHR)], sw[b]
                ).wait()

    return gather_kernel


_gather = _make_gather()


def kernel(sentence, table):
    idx = jnp.swapaxes(sentence, 0, 1).reshape(B).astype(jnp.int32)
    out_q = _gather(idx, table)
    return out_q.transpose(2, 4, 0, 1, 3).reshape(ROWS, COLS, EMBED_DIM)


# speedup vs baseline: 1.2304x; 1.2304x over previous
import functools

import jax
import jax.numpy as jnp
from jax import lax
from jax.experimental import pallas as pl
from jax.experimental.pallas import tpu as pltpu
from jax.experimental.pallas import tpu_sc as plsc

ROWS = 4096
COLS = 200
EMBED_DIM = 32
B = ROWS * COLS

_NUM_CORES = 2
_NUM_SUBCORES = 16
NW = _NUM_CORES * _NUM_SUBCORES

UNIT = 512
IHR = UNIT // 128
NDH = EMBED_DIM // 8
UNITS_PER_COL = ROWS // UNIT
NUNITS = COLS * UNITS_PER_COL
UNITS_PER_W = NUNITS // NW


def _make_gather():
    mesh = plsc.VectorSubcoreMesh(core_axis_name="c", subcore_axis_name="s")

    @functools.partial(
        pl.kernel,
        mesh=mesh,
        out_type=jax.ShapeDtypeStruct((COLS, NDH, ROWS // 128, 8, 128),
                                      jnp.float32),
        compiler_params=pltpu.CompilerParams(
            use_tc_tiling_on_sc=False, needs_layout_passes=False,
            disable_bounds_checks=True),
        scratch_types=[
            pltpu.VMEM((UNIT,), jnp.int32),
            pltpu.VMEM((UNIT,), jnp.int32),
            pltpu.VMEM((UNIT, EMBED_DIM), jnp.float32),
            pltpu.VMEM((UNIT, EMBED_DIM), jnp.float32),
            pltpu.VMEM((NDH, IHR, 8, 128), jnp.float32),
            pltpu.VMEM((NDH, IHR, 8, 128), jnp.float32),
            pltpu.SemaphoreType.DMA,
            pltpu.SemaphoreType.DMA,
            pltpu.SemaphoreType.DMA,
            pltpu.SemaphoreType.DMA,
        ],
    )
    def gather_kernel(idx_hbm, table_hbm, out5_hbm, idx0, idx1, g0, g1,
                      t0, t1, sg0, sg1, sw0, sw1):
        wid = lax.axis_index("s") * _NUM_CORES + lax.axis_index("c")
        u0 = wid * UNITS_PER_W
        idx_b = (idx0, idx1)
        g_b = (g0, g1)
        t_b = (t0, t1)
        sg = (sg0, sg1)
        sw = (sw0, sw1)
        iota = lax.iota(jnp.int32, 16)
        iotas = [iota + il8 * 16 for il8 in range(8)]

        def idx_off(u):
            gu = u0 + u
            return (gu // UNITS_PER_COL) * ROWS + (gu % UNITS_PER_COL) * UNIT

        pltpu.sync_copy(idx_hbm.at[pl.ds(idx_off(0), UNIT)], idx0)
        pltpu.async_copy(table_hbm.at[idx0], g0, sg0)

        def unit_step(u, b):
            gu = u0 + u
            j = gu // UNITS_PER_COL
            ir = gu % UNITS_PER_COL
            gbuf = g_b[b]
            tbuf = t_b[b]

            pltpu.make_async_copy(table_hbm.at[idx_b[b]], gbuf, sg[b]).wait()

            @pl.when(u + 1 < UNITS_PER_W)
            def _prefetch():
                pltpu.sync_copy(
                    idx_hbm.at[pl.ds(idx_off(u + 1), UNIT)], idx_b[1 - b])
                pltpu.async_copy(
                    table_hbm.at[idx_b[1 - b]], g_b[1 - b], sg[1 - b])

            @pl.when(u >= 2)
            def _drain():
                for _ in range(NDH):
                    pltpu.make_async_copy(
                        tbuf.at[0], out5_hbm.at[0, 0, pl.ds(0, IHR)], sw[b]
                    ).wait()

            @plsc.parallel_loop(0, NDH * IHR, unroll=2)
            def _transpose(k):
                dh = k // IHR
                ihr = k % IHR
                mbase = ihr * 128
                for il8 in range(8):
                    mvec = mbase + iotas[il8]
                    for dl in range(8):
                        dvec = jnp.full((16,), 0, jnp.int32) + (dh * 8 + dl)
                        vals = plsc.load_gather(gbuf, [mvec, dvec])
                        tbuf[dh, ihr, dl, pl.ds(il8 * 16, 16)] = vals

            for dh in range(NDH):
                pltpu.async_copy(
                    tbuf.at[dh],
                    out5_hbm.at[j, dh, pl.ds(ir * IHR, IHR)],
                    sw[b],
                )

        def outer(g2, _):
            for b in range(2):
                unit_step(g2 * 2 + b, b)
            return 0

        lax.fori_loop(0, UNITS_PER_W // 2, outer, 0)

        for b in range(2):
            for _ in range(NDH):
                pltpu.make_async_copy(
                    t_b[b].at[0], out5_hbm.at[0, 0, pl.ds(0, IHR)], sw[b]
                ).wait()

    return gather_kernel


_gather = _make_gather()


def kernel(sentence, table):
    idx = jnp.swapaxes(sentence, 0, 1).reshape(B).astype(jnp.int32)
    out_q = _gather(idx, table)
    return out_q.transpose(2, 4, 0, 1, 3).reshape(ROWS, COLS, EMBED_DIM)
